# baseline (speedup 1.0000x reference)
import functools

import jax
import jax.numpy as jnp
from jax.experimental import pallas as pl
from jax.experimental.pallas import tpu as pltpu

_B, _S, _DFF = 4, 2048, 4096
_K = 32
_ROWS = _B * _S


def _row_sum(m, ones_col):
    return jax.lax.dot_general(
        m, ones_col, (((1,), (0,)), ((), ())),
        preferred_element_type=jnp.float32)


def _gated_gelu_body(x_ref, la_ref, ls_ref, mean_ref, sq_ref, out_ref):
    xb = x_ref[...]
    mean = mean_ref[...]
    var = jnp.maximum(sq_ref[...] - mean * mean, 1e-6)
    inv_std = jax.lax.rsqrt(var)
    z = jnp.abs(xb - mean) * inv_std

    ones_col = jnp.ones((_DFF, 1), jnp.float32)

    t = jnp.zeros((xb.shape[0], 1), jnp.int32)
    for b in range(30, -1, -1):
        cand = t | (1 << b)
        cand_f = jax.lax.bitcast_convert_type(cand, jnp.float32)
        ind = jnp.where(z >= cand_f, 1.0, 0.0)
        cnt = _row_sum(ind, ones_col)
        t = jnp.where(cnt >= _K, cand, t)
    tf = jax.lax.bitcast_convert_type(t, jnp.float32)

    gt = z > tf
    cnt_gt = _row_sum(jnp.where(gt, 1.0, 0.0), ones_col)
    sum_gt = _row_sum(jnp.where(gt, z, 0.0), ones_col)
    surp = (sum_gt + (_K - cnt_gt) * tf) * (1.0 / _K)

    alpha = jnp.exp(la_ref[0, 0])
    sigma = jnp.exp(ls_ref[0, 0])
    gate = 1.0 + alpha * jnp.tanh(sigma * surp)

    base = 0.5 * xb * (1.0 + jax.lax.erf(xb * 0.7071067811865476))
    out_ref[...] = base * gate


@jax.jit
def kernel(x, log_alpha, log_sigma, ema_mean, ema_sq):
    xf = x.reshape(_ROWS, _DFF)
    rows_per_block = 256
    grid = (_ROWS // rows_per_block,)
    la = log_alpha.reshape(1, 1)
    ls = log_sigma.reshape(1, 1)
    mean2d = ema_mean.reshape(1, _DFF)
    sq2d = ema_sq.reshape(1, _DFF)
    out = pl.pallas_call(
        _gated_gelu_body,
        grid=grid,
        in_specs=[
            pl.BlockSpec((rows_per_block, _DFF), lambda i: (i, 0)),
            pl.BlockSpec(memory_space=pltpu.SMEM),
            pl.BlockSpec(memory_space=pltpu.SMEM),
            pl.BlockSpec((1, _DFF), lambda i: (0, 0)),
            pl.BlockSpec((1, _DFF), lambda i: (0, 0)),
        ],
        out_specs=pl.BlockSpec((rows_per_block, _DFF), lambda i: (i, 0)),
        out_shape=jax.ShapeDtypeStruct((_ROWS, _DFF), jnp.float32),
    )(xf, la, ls, mean2d, sq2d)
    return out.reshape(_B, _S, _DFF)


# speedup vs baseline: 3.3201x; 3.3201x over previous
import functools

import jax
import jax.numpy as jnp
from jax.experimental import pallas as pl
from jax.experimental.pallas import tpu as pltpu

_B, _S, _DFF = 4, 2048, 4096
_K = 32
_ROWS = _B * _S


def _gated_gelu_body(x_ref, la_ref, ls_ref, mean_ref, sq_ref, out_ref):
    xb = x_ref[...]
    mean = mean_ref[...]
    var = jnp.maximum(sq_ref[...] - mean * mean, 1e-6)
    inv_std = jax.lax.rsqrt(var)
    z = jnp.abs(xb - mean) * inv_std

    t = jnp.zeros((xb.shape[0], 1), jnp.int32)
    for b in range(30, 8, -1):
        cand = t | (1 << b)
        cand_f = jax.lax.bitcast_convert_type(cand, jnp.float32)
        cnt = jnp.sum((z >= cand_f).astype(jnp.float32), axis=-1,
                      keepdims=True)
        t = jnp.where(cnt >= _K, cand, t)
    tf = jax.lax.bitcast_convert_type(t, jnp.float32)

    gt = z > tf
    cnt_gt = jnp.sum(gt.astype(jnp.float32), axis=-1, keepdims=True)
    sum_gt = jnp.sum(jnp.where(gt, z, 0.0), axis=-1, keepdims=True)
    surp = (sum_gt + (_K - cnt_gt) * tf) * (1.0 / _K)

    alpha = jnp.exp(la_ref[0, 0])
    sigma = jnp.exp(ls_ref[0, 0])
    gate = 1.0 + alpha * jnp.tanh(sigma * surp)

    base = 0.5 * xb * (1.0 + jax.lax.erf(xb * 0.7071067811865476))
    out_ref[...] = base * gate


@jax.jit
def kernel(x, log_alpha, log_sigma, ema_mean, ema_sq):
    xf = x.reshape(_ROWS, _DFF)
    rows_per_block = 256
    grid = (_ROWS // rows_per_block,)
    la = log_alpha.reshape(1, 1)
    ls = log_sigma.reshape(1, 1)
    mean2d = ema_mean.reshape(1, _DFF)
    sq2d = ema_sq.reshape(1, _DFF)
    out = pl.pallas_call(
        _gated_gelu_body,
        grid=grid,
        in_specs=[
            pl.BlockSpec((rows_per_block, _DFF), lambda i: (i, 0)),
            pl.BlockSpec(memory_space=pltpu.SMEM),
            pl.BlockSpec(memory_space=pltpu.SMEM),
            pl.BlockSpec((1, _DFF), lambda i: (0, 0)),
            pl.BlockSpec((1, _DFF), lambda i: (0, 0)),
        ],
        out_specs=pl.BlockSpec((rows_per_block, _DFF), lambda i: (i, 0)),
        out_shape=jax.ShapeDtypeStruct((_ROWS, _DFF), jnp.float32),
    )(xf, la, ls, mean2d, sq2d)
    return out.reshape(_B, _S, _DFF)
